# baseline (speedup 1.0000x reference)
import functools

import numpy as np
import jax
import jax.numpy as jnp
from jax.experimental import pallas as pl
from jax.experimental.pallas import tpu as pltpu
from jax.experimental.pallas import tpu_sc as plsc

_CHUNK = 8


def kernel(inputs, relative_embedding):
    seq_len = inputs.shape[1]
    num_rows, d = relative_embedding.shape
    max_d = (num_rows - 1) // 2

    idx = np.arange(-max_d, max_d + 1)
    rel = np.clip(idx, -seq_len + 1, seq_len - 1) + max_d
    runs = []
    start = 0
    for i in range(1, num_rows + 1):
        if i == num_rows or rel[i] != rel[i - 1] + 1:
            runs.append((start, int(rel[start]), i - start))
            start = i
    chunks = []
    for out0, tab0, n in runs:
        off = 0
        while off < n:
            ln = min(_CHUNK - (out0 + off) % _CHUNK, n - off)
            chunks.append((out0 + off, tab0 + off, ln))
            off += ln

    mesh = plsc.ScalarSubcoreMesh(axis_name="c", num_cores=1)

    @functools.partial(
        pl.kernel,
        mesh=mesh,
        out_type=jax.ShapeDtypeStruct((num_rows, d), jnp.float32),
        scratch_types=[pltpu.SemaphoreType.DMA],
    )
    def emb_gather(table_hbm, out_hbm, sem):
        copies = [
            pltpu.async_copy(
                table_hbm.at[pl.ds(tab0, ln)], out_hbm.at[pl.ds(out0, ln)], sem
            )
            for out0, tab0, ln in chunks
        ]
        for cp in copies:
            cp.wait()

    return emb_gather(relative_embedding)


# speedup vs baseline: 1.1196x; 1.1196x over previous
import functools

import jax
import jax.numpy as jnp
from jax import lax
from jax.experimental import pallas as pl
from jax.experimental.pallas import tpu as pltpu
from jax.experimental.pallas import tpu_sc as plsc

_LANES = 16
_CHUNK = 8


def kernel(inputs, relative_embedding):
    seq_len = inputs.shape[1]
    num_rows, d = relative_embedding.shape
    max_d = (num_rows - 1) // 2
    lo, hi = -seq_len + 1, seq_len - 1

    n_full = num_rows // _CHUNK
    rem = num_rows - n_full * _CHUNK

    mesh = plsc.VectorSubcoreMesh(
        core_axis_name="c", subcore_axis_name="s", num_cores=2
    )

    n_row_chunks = n_full + (1 if rem else 0)
    dh = d // 4

    @functools.partial(
        pl.kernel,
        mesh=mesh,
        out_type=jax.ShapeDtypeStruct((num_rows, d), jnp.float32),
        scratch_types=[
            pltpu.VMEM((_LANES,), jnp.int32),
            pltpu.VMEM((_CHUNK, dh), jnp.float32),
            pltpu.SemaphoreType.DMA,
        ],
    )
    def emb_gather(table_hbm, out_hbm, idx_v, rows_v, sem):
        wid = lax.axis_index("s") * 2 + lax.axis_index("c")
        rc = wid // 4
        coff = (wid % 4) * dh
        base = rc * _CHUNK

        p = lax.iota(jnp.int32, _LANES) + base
        r = jnp.minimum(jnp.maximum(p - max_d, lo), hi) + max_d
        idx_v[...] = jnp.minimum(r, num_rows - 1)

        @pl.when(rc < n_full)
        def _full():
            pltpu.async_copy(
                table_hbm.at[idx_v.at[pl.ds(0, _CHUNK)], pl.ds(coff, dh)],
                rows_v,
                sem,
            ).wait()
            pltpu.sync_copy(
                rows_v, out_hbm.at[pl.ds(base, _CHUNK), pl.ds(coff, dh)]
            )

        if rem:

            @pl.when(rc == n_full)
            def _tail():
                pltpu.async_copy(
                    table_hbm.at[idx_v.at[pl.ds(0, _CHUNK)], pl.ds(coff, dh)],
                    rows_v,
                    sem,
                ).wait()
                pltpu.sync_copy(
                    rows_v.at[pl.ds(0, rem)],
                    out_hbm.at[pl.ds(base, rem), pl.ds(coff, dh)],
                )

    return emb_gather(relative_embedding)


# speedup vs baseline: 1.1946x; 1.0670x over previous
import functools

import jax
import jax.numpy as jnp
from jax import lax
from jax.experimental import pallas as pl
from jax.experimental.pallas import tpu as pltpu
from jax.experimental.pallas import tpu_sc as plsc

_LANES = 16
_CHUNK = 8


def kernel(inputs, relative_embedding):
    seq_len = inputs.shape[1]
    num_rows, d = relative_embedding.shape
    max_d = (num_rows - 1) // 2
    lo, hi = -seq_len + 1, seq_len - 1

    n_full = num_rows // _CHUNK
    rem = num_rows - n_full * _CHUNK

    mesh = plsc.VectorSubcoreMesh(
        core_axis_name="c", subcore_axis_name="s", num_cores=1
    )

    n_row_chunks = n_full + (1 if rem else 0)
    dh = d // 2

    @functools.partial(
        pl.kernel,
        mesh=mesh,
        out_type=jax.ShapeDtypeStruct((num_rows, d), jnp.float32),
        scratch_types=[
            pltpu.VMEM((_LANES,), jnp.int32),
            pltpu.VMEM((_CHUNK, dh), jnp.float32),
            pltpu.SemaphoreType.DMA,
        ],
    )
    def emb_gather(table_hbm, out_hbm, idx_v, rows_v, sem):
        wid = lax.axis_index("s")
        rc = wid // 2
        coff = (wid % 2) * dh
        base = rc * _CHUNK

        p = lax.iota(jnp.int32, _LANES) + base
        r = jnp.minimum(jnp.maximum(p - max_d, lo), hi) + max_d
        idx_v[...] = jnp.minimum(r, num_rows - 1)

        @pl.when(rc < n_full)
        def _full():
            pltpu.async_copy(
                table_hbm.at[idx_v.at[pl.ds(0, _CHUNK)], pl.ds(coff, dh)],
                rows_v,
                sem,
            ).wait()
            pltpu.sync_copy(
                rows_v, out_hbm.at[pl.ds(base, _CHUNK), pl.ds(coff, dh)]
            )

        if rem:

            @pl.when(rc == n_full)
            def _tail():
                pltpu.async_copy(
                    table_hbm.at[idx_v.at[pl.ds(0, _CHUNK)], pl.ds(coff, dh)],
                    rows_v,
                    sem,
                ).wait()
                pltpu.sync_copy(
                    rows_v.at[pl.ds(0, rem)],
                    out_hbm.at[pl.ds(base, rem), pl.ds(coff, dh)],
                )

    return emb_gather(relative_embedding)
